# baseline (speedup 1.0000x reference)
import jax
import jax.numpy as jnp
from jax.experimental import pallas as pl
import jax.experimental.pallas.tpu as pltpu

_B = 16
_C = 768
_P = 512
_HW = 1024
_KI = 4
_KO = 8


def _concat_pos_kernel(x_hbm, row_ref, col_ref, o_hbm, cbuf, tile, in_sems, out_sems):
    colb = jnp.broadcast_to(col_ref[...][None, :, :], (32, 32, 256)).reshape(_HW, 256)
    rowb = jnp.broadcast_to(row_ref[...][:, None, :], (32, 32, 256)).reshape(_HW, 256)
    for s in range(_KO):
        tile[s, :, _C:_C + 256] = colb
        tile[s, :, _C + 256:] = rowb

    def in_copy(i):
        return pltpu.make_async_copy(x_hbm.at[i], cbuf.at[i % _KI], in_sems.at[i % _KI])

    out_copies = [
        pltpu.make_async_copy(tile.at[i % _KO], o_hbm.at[i], out_sems.at[i % _KO])
        for i in range(_B)
    ]

    for i in range(_KI):
        in_copy(i).start()
    for i in range(_B):
        in_copy(i).wait()
        if i >= _KO:
            out_copies[i - _KO].wait()
        tile[i % _KO, :, 0:_C] = cbuf[i % _KI]
        out_copies[i].start()
        if i + _KI < _B:
            in_copy(i + _KI).start()
    for i in range(_B - _KO, _B):
        out_copies[i].wait()


def kernel(x, row_embed, col_embed):
    b, c, h, w = x.shape
    xt = x.transpose(0, 2, 3, 1).reshape(b, h * w, c)
    out = pl.pallas_call(
        _concat_pos_kernel,
        in_specs=[
            pl.BlockSpec(memory_space=pl.ANY),
            pl.BlockSpec(memory_space=pltpu.MemorySpace.VMEM),
            pl.BlockSpec(memory_space=pltpu.MemorySpace.VMEM),
        ],
        out_specs=pl.BlockSpec(memory_space=pl.ANY),
        out_shape=jax.ShapeDtypeStruct((b, h * w, c + _P), x.dtype),
        scratch_shapes=[
            pltpu.VMEM((_KI, h * w, c), x.dtype),
            pltpu.VMEM((_KO, h * w, c + _P), x.dtype),
            pltpu.SemaphoreType.DMA((_KI,)),
            pltpu.SemaphoreType.DMA((_KO,)),
        ],
    )(xt, row_embed, col_embed)
    return out.reshape(b, h, w, c + _P).transpose(0, 3, 1, 2)


# speedup vs baseline: 1.0010x; 1.0010x over previous
import jax
import jax.numpy as jnp
from jax.experimental import pallas as pl
import jax.experimental.pallas.tpu as pltpu

_B = 16
_C = 768
_P = 512
_HW = 1024
_KI = 4
_KO = 6


def _concat_pos_kernel(x_hbm, row_ref, col_ref, o_hbm, cbuf, tile, in_sems, out_sems):
    colb = jnp.broadcast_to(col_ref[...][None, :, :], (32, 32, 256)).reshape(_HW, 256)
    rowb = jnp.broadcast_to(row_ref[...][:, None, :], (32, 32, 256)).reshape(_HW, 256)
    for s in range(_KO):
        tile[s, :, _C:_C + 256] = colb
        tile[s, :, _C + 256:] = rowb

    def in_copy(i):
        return pltpu.make_async_copy(x_hbm.at[i], cbuf.at[i % _KI], in_sems.at[i % _KI])

    out_copies = [
        pltpu.make_async_copy(tile.at[i % _KO], o_hbm.at[i], out_sems.at[i % _KO])
        for i in range(_B)
    ]

    for i in range(_KI):
        in_copy(i).start()
    for i in range(_B):
        in_copy(i).wait()
        if i >= _KO:
            out_copies[i - _KO].wait()
        tile[i % _KO, :, 0:_C] = cbuf[i % _KI]
        out_copies[i].start()
        if i + _KI < _B:
            in_copy(i + _KI).start()
    for i in range(_B - _KO, _B):
        out_copies[i].wait()


def kernel(x, row_embed, col_embed):
    b, c, h, w = x.shape
    xt = x.transpose(0, 2, 3, 1).reshape(b, h * w, c)
    out = pl.pallas_call(
        _concat_pos_kernel,
        in_specs=[
            pl.BlockSpec(memory_space=pl.ANY),
            pl.BlockSpec(memory_space=pltpu.MemorySpace.VMEM),
            pl.BlockSpec(memory_space=pltpu.MemorySpace.VMEM),
        ],
        out_specs=pl.BlockSpec(memory_space=pl.ANY),
        out_shape=jax.ShapeDtypeStruct((b, h * w, c + _P), x.dtype),
        scratch_shapes=[
            pltpu.VMEM((_KI, h * w, c), x.dtype),
            pltpu.VMEM((_KO, h * w, c + _P), x.dtype),
            pltpu.SemaphoreType.DMA((_KI,)),
            pltpu.SemaphoreType.DMA((_KO,)),
        ],
    )(xt, row_embed, col_embed)
    return out.reshape(b, h, w, c + _P).transpose(0, 3, 1, 2)
